# baseline (speedup 1.0000x reference)
import math

import jax
import jax.numpy as jnp
from jax import lax
from jax.experimental import pallas as pl
from jax.experimental.pallas import tpu as pltpu
from jax.experimental.pallas import tpu_sc as plsc

D_MODEL = 64
VOCAB = 1000000
BATCH = 4096
SEQ = 200
SCALE = math.sqrt(D_MODEL)

NC = 2
NS = 16
NW = NC * NS

ROWS_PER_W = BATCH // NW
CHUNK_ROWS = 2
N_CHUNKS = ROWS_PER_W // CHUNK_ROWS
SEQ_SPLITS = ((0, 128), (128, 72))


def _sc_body(x_hbm, table_hbm, out_hbm, idx_v, rows_v, s_g0, s_g1, s_o0, s_o1):
    s_g = (s_g0, s_g1)
    s_o = (s_o0, s_o1)
    wid = lax.axis_index("s") * NC + lax.axis_index("c")
    row_base = wid * ROWS_PER_W

    def fire_gathers(c, b):
        for r in range(CHUNK_ROWS):
            for off, n in SEQ_SPLITS:
                pltpu.async_copy(
                    table_hbm.at[idx_v.at[c * CHUNK_ROWS + r, pl.ds(off, n)]],
                    rows_v.at[b, r, pl.ds(off, n)],
                    s_g[b],
                )

    def wait_gathers(b):
        for r in range(CHUNK_ROWS):
            for off, n in SEQ_SPLITS:
                pltpu.make_async_copy(
                    table_hbm.at[idx_v.at[r, pl.ds(off, n)]],
                    rows_v.at[b, r, pl.ds(off, n)],
                    s_g[b],
                ).wait()

    def wait_store(b):
        pltpu.make_async_copy(
            rows_v.at[b], out_hbm.at[pl.ds(row_base, CHUNK_ROWS)], s_o[b]
        ).wait()

    pltpu.sync_copy(x_hbm.at[pl.ds(row_base, ROWS_PER_W)], idx_v)
    fire_gathers(0, 0)

    def step(c, b):
        @pl.when(c + 1 < N_CHUNKS)
        def _fire_next():
            nb = 1 - b
            @pl.when(c >= 1)
            def _drain_prev_store():
                wait_store(nb)

            fire_gathers(c + 1, nb)

        wait_gathers(b)

        for r in range(CHUNK_ROWS):
            @plsc.parallel_loop(0, SEQ, unroll=8)
            def _scale(s):
                for col in range(D_MODEL // 16):
                    sl = pl.ds(col * 16, 16)
                    rows_v[b, r, s, sl] = rows_v[b, r, s, sl] * SCALE

        pltpu.async_copy(
            rows_v.at[b],
            out_hbm.at[pl.ds(row_base + c * CHUNK_ROWS, CHUNK_ROWS)],
            s_o[b],
        )

    def pair(i, carry):
        step(i * 2, 0)
        step(i * 2 + 1, 1)
        return carry

    lax.fori_loop(0, N_CHUNKS // 2, pair, 0)
    wait_store(0)
    wait_store(1)


@jax.jit
def _embed(x, lut_weight):
    mesh = plsc.VectorSubcoreMesh(core_axis_name="c", subcore_axis_name="s")
    return pl.kernel(
        _sc_body,
        out_type=jax.ShapeDtypeStruct((BATCH, SEQ, D_MODEL), jnp.float32),
        mesh=mesh,
        scratch_types=[
            pltpu.VMEM((ROWS_PER_W, SEQ), jnp.int32),
            pltpu.VMEM((2, CHUNK_ROWS, SEQ, D_MODEL), jnp.float32),
            pltpu.SemaphoreType.DMA,
            pltpu.SemaphoreType.DMA,
            pltpu.SemaphoreType.DMA,
            pltpu.SemaphoreType.DMA,
        ],
        compiler_params=pltpu.CompilerParams(use_tc_tiling_on_sc=False),
    )(x, lut_weight)


def kernel(x, lut_weight):
    return _embed(x, lut_weight)
